# Initial kernel scaffold; baseline (speedup 1.0000x reference)
import jax
import jax.numpy as jnp
from jax.experimental import pallas as pl


def kernel(x, edge_index, W1, b1, W2, b2):
    raise NotImplementedError("write your pallas kernel here")


# speedup vs baseline: 35.7107x; 35.7107x over previous
import functools

import jax
import jax.numpy as jnp
from jax import lax
from jax.experimental import pallas as pl
from jax.experimental.pallas import tpu as pltpu
from jax.experimental.pallas import tpu_sc as plsc

N = 10000
H = 128
HH = H // 2
C = 16

NC = 2
NS = 16
NW = NC * NS
K = 128

ROWS_PER_TILE = 632
NACC = NS * ROWS_PER_TILE
ROWS_DEG = 640
NACC_DEG = NS * ROWS_DEG


def _mesh():
    return plsc.VectorSubcoreMesh(core_axis_name="c", subcore_axis_name="s")


_SC_PARAMS = pltpu.CompilerParams(use_tc_tiling_on_sc=False)


@functools.lru_cache(maxsize=None)
def _deg_kernel(J):
    @functools.partial(
        pl.kernel,
        out_type=jax.ShapeDtypeStruct((NC, NACC_DEG), jnp.float32),
        mesh=_mesh(),
        compiler_params=_SC_PARAMS,
        scratch_types=[
            pltpu.VMEM((J, K), jnp.int32),
            pltpu.VMEM((K,), jnp.float32),
            pltpu.VMEM((ROWS_DEG,), jnp.float32),
            pltpu.VMEM_SHARED((NACC_DEG,), jnp.float32),
        ],
    )
    def deg(dst_hbm, out_hbm, dst_v, ones_v, buf_v, acc):
        c = lax.axis_index("c")
        s = lax.axis_index("s")
        w = c * NS + s
        for i in range(K // 16):
            ones_v[pl.ds(i * 16, 16)] = jnp.full((16,), 1.0, jnp.float32)

        def zero_body(i, carry):
            buf_v[pl.ds(i * 16, 16)] = jnp.zeros((16,), jnp.float32)
            return carry

        lax.fori_loop(0, ROWS_DEG // 16, zero_body, 0)
        base = s * ROWS_DEG
        pltpu.sync_copy(buf_v, acc.at[pl.ds(base, ROWS_DEG)])
        pltpu.sync_copy(dst_hbm.at[w], dst_v)
        plsc.subcore_barrier()

        def body(j, carry):
            pltpu.sync_copy(ones_v, acc.at[dst_v.at[j]], add=True)
            return carry

        lax.fori_loop(0, J, body, 0)
        plsc.subcore_barrier()
        pltpu.sync_copy(acc.at[pl.ds(base, ROWS_DEG)], buf_v)
        pltpu.sync_copy(buf_v, out_hbm.at[c, pl.ds(base, ROWS_DEG)])

    return deg


def _gather_scatter_loop(y_ref, src_v, dst_v, rows_a, rows_b, acc, sem_a,
                         sem_b, J):
    pltpu.async_copy(y_ref.at[src_v.at[0]], rows_a, sem_a)

    def pair(t, carry):
        j0 = 2 * t
        pltpu.async_copy(y_ref.at[src_v.at[j0 + 1]], rows_b, sem_b)
        pltpu.make_async_copy(y_ref.at[src_v.at[j0]], rows_a, sem_a).wait()
        pltpu.sync_copy(rows_a, acc.at[dst_v.at[j0]], add=True)
        pltpu.async_copy(y_ref.at[src_v.at[j0 + 2]], rows_a, sem_a)
        pltpu.make_async_copy(y_ref.at[src_v.at[j0 + 1]], rows_b, sem_b).wait()
        pltpu.sync_copy(rows_b, acc.at[dst_v.at[j0 + 1]], add=True)
        return carry

    lax.fori_loop(0, (J - 1) // 2, pair, 0)
    pltpu.make_async_copy(y_ref.at[src_v.at[J - 1]], rows_a, sem_a).wait()
    pltpu.sync_copy(rows_a, acc.at[dst_v.at[J - 1]], add=True)


def _zero_acc_slice(rows_a, acc, base, D):
    def zrow(i, carry):
        for jj in range(D // 16):
            rows_a[i, pl.ds(jj * 16, 16)] = jnp.zeros((16,), jnp.float32)
        return carry

    lax.fori_loop(0, K, zrow, 0)
    nfull = ROWS_PER_TILE // K
    rem = ROWS_PER_TILE - nfull * K
    for t in range(nfull):
        pltpu.sync_copy(rows_a, acc.at[pl.ds(base + t * K, K)])
    pltpu.sync_copy(rows_a.at[pl.ds(0, rem)],
                    acc.at[pl.ds(base + nfull * K, rem)])


def _copy_out_slice(rows_a, acc, out_ref, base, D):
    nfull = ROWS_PER_TILE // K
    rem = ROWS_PER_TILE - nfull * K
    for t in range(nfull):
        pltpu.sync_copy(acc.at[pl.ds(base + t * K, K)], rows_a)
        pltpu.sync_copy(rows_a, out_ref.at[pl.ds(base + t * K, K)])
    pltpu.sync_copy(acc.at[pl.ds(base + nfull * K, rem)],
                    rows_a.at[pl.ds(0, rem)])
    pltpu.sync_copy(rows_a.at[pl.ds(0, rem)],
                    out_ref.at[pl.ds(base + nfull * K, rem)])


@functools.lru_cache(maxsize=None)
def _agg1_kernel(J):
    @functools.partial(
        pl.kernel,
        out_type=jax.ShapeDtypeStruct((NC, NACC, HH), jnp.float32),
        mesh=_mesh(),
        compiler_params=_SC_PARAMS,
        scratch_types=[
            pltpu.VMEM((J, K), jnp.int32),
            pltpu.VMEM((J, K), jnp.int32),
            pltpu.VMEM((K, HH), jnp.float32),
            pltpu.VMEM((K, HH), jnp.float32),
            pltpu.VMEM_SHARED((NACC, HH), jnp.float32),
            pltpu.SemaphoreType.DMA,
            pltpu.SemaphoreType.DMA,
        ],
    )
    def agg1(y_hbm, src_hbm, dst_hbm, out_hbm, src_v, dst_v, rows_a, rows_b,
             acc, sem_a, sem_b):
        c = lax.axis_index("c")
        s = lax.axis_index("s")
        base = s * ROWS_PER_TILE
        _zero_acc_slice(rows_a, acc, base, HH)
        pltpu.sync_copy(src_hbm.at[s], src_v)
        pltpu.sync_copy(dst_hbm.at[s], dst_v)
        plsc.subcore_barrier()
        _gather_scatter_loop(y_hbm.at[c], src_v, dst_v, rows_a, rows_b, acc,
                             sem_a, sem_b, J)
        plsc.subcore_barrier()
        _copy_out_slice(rows_a, acc, out_hbm.at[c], base, HH)

    return agg1


@functools.lru_cache(maxsize=None)
def _agg2_kernel(J):
    @functools.partial(
        pl.kernel,
        out_type=jax.ShapeDtypeStruct((NC, NACC, C), jnp.float32),
        mesh=_mesh(),
        compiler_params=_SC_PARAMS,
        scratch_types=[
            pltpu.VMEM((J, K), jnp.int32),
            pltpu.VMEM((J, K), jnp.int32),
            pltpu.VMEM((K, C), jnp.float32),
            pltpu.VMEM((K, C), jnp.float32),
            pltpu.VMEM_SHARED((NACC, C), jnp.float32),
            pltpu.SemaphoreType.DMA,
            pltpu.SemaphoreType.DMA,
        ],
    )
    def agg2(y_hbm, src_hbm, dst_hbm, out_hbm, src_v, dst_v, rows_a, rows_b,
             acc, sem_a, sem_b):
        c = lax.axis_index("c")
        s = lax.axis_index("s")
        w = c * NS + s
        base = s * ROWS_PER_TILE
        _zero_acc_slice(rows_a, acc, base, C)
        pltpu.sync_copy(src_hbm.at[w], src_v)
        pltpu.sync_copy(dst_hbm.at[w], dst_v)
        plsc.subcore_barrier()
        _gather_scatter_loop(y_hbm, src_v, dst_v, rows_a, rows_b, acc,
                             sem_a, sem_b, J)
        plsc.subcore_barrier()
        _copy_out_slice(rows_a, acc, out_hbm.at[c], base, C)

    return agg2


def _prep(x, W1, degp):
    def body(x_ref, w_ref, deg_ref, yh_ref, dinv_ref):
        dinv = lax.rsqrt(deg_ref[0] + deg_ref[1] + 1.0)
        xw = jnp.dot(x_ref[...], w_ref[...], preferred_element_type=jnp.float32)
        y = xw * dinv
        yh_ref[0] = y[:, :HH]
        yh_ref[1] = y[:, HH:]
        dinv_ref[...] = dinv

    return pl.pallas_call(
        body,
        out_shape=(
            jax.ShapeDtypeStruct((NC, N, HH), jnp.float32),
            jax.ShapeDtypeStruct((N, 1), jnp.float32),
        ),
    )(x, W1, degp)


def _mid(agg1, yh, dinv, b1, W2):
    def body(agg_ref, yh_ref, dinv_ref, b_ref, w_ref, o_ref):
        dinv = dinv_ref[...]
        acat = jnp.concatenate([agg_ref[0, :N], agg_ref[1, :N]], axis=1)
        ycat = jnp.concatenate([yh_ref[0], yh_ref[1]], axis=1)
        pre = dinv * (acat + ycat) + b_ref[...][None, :]
        h = jnp.maximum(pre, 0.0)
        o_ref[...] = dinv * jnp.dot(h, w_ref[...],
                                    preferred_element_type=jnp.float32)

    return pl.pallas_call(
        body,
        out_shape=jax.ShapeDtypeStruct((N, C), jnp.float32),
    )(agg1, yh, dinv, b1, W2)


def _final(agg2, y2, dinv, b2):
    def body(agg_ref, y_ref, dinv_ref, b_ref, o_ref):
        o_ref[...] = (dinv_ref[...] * (agg_ref[0, :N] + agg_ref[1, :N]
                                       + y_ref[...])
                      + b_ref[...][None, :])

    return pl.pallas_call(
        body,
        out_shape=jax.ShapeDtypeStruct((N, C), jnp.float32),
    )(agg2, y2, dinv, b2)


def _pad_edges(ei, shards, J):
    E = ei.shape[1]
    pad = shards * J * K - E
    ar = jnp.arange(pad, dtype=jnp.int32)
    pad_src = (ar * 7993) % N
    pad_dst = N + (ar % (NACC - N))
    src = jnp.concatenate([ei[0], pad_src]).reshape(shards, J, K)
    dst = jnp.concatenate([ei[1], pad_dst]).reshape(shards, J, K)
    return src, dst


def _num_chunks(E, shards):
    J = -(-E // (shards * K))
    if J % 2 == 0:
        J += 1
    return J


def kernel(x, edge_index, W1, b1, W2, b2):
    E = edge_index.shape[1]
    ei = edge_index.astype(jnp.int32)

    J1 = _num_chunks(E, NS)
    J2 = _num_chunks(E, NW)
    src1, dst1 = _pad_edges(ei, NS, J1)
    src2, dst2 = _pad_edges(ei, NW, J2)

    deg = _deg_kernel(J2)(dst2)
    degp = deg[:, :N, None]
    yh, dinv = _prep(x, W1, degp)
    agg1 = _agg1_kernel(J1)(yh, src1, dst1)
    y2 = _mid(agg1, yh, dinv, b1, W2)
    agg2 = _agg2_kernel(J2)(y2, src2, dst2)
    return _final(agg2, y2, dinv, b2)


# speedup vs baseline: 40.2667x; 1.1276x over previous
import functools

import jax
import jax.numpy as jnp
from jax import lax
from jax.experimental import pallas as pl
from jax.experimental.pallas import tpu as pltpu
from jax.experimental.pallas import tpu_sc as plsc

N = 10000
H = 128
HH = H // 2
C = 16

NC = 2
NS = 16
NW = NC * NS
K = 128

ROWS_PER_TILE = 632
NACC = NS * ROWS_PER_TILE
ROWS_DEG = 640
NACC_DEG = NS * ROWS_DEG


def _mesh():
    return plsc.VectorSubcoreMesh(core_axis_name="c", subcore_axis_name="s")


_SC_PARAMS = pltpu.CompilerParams(use_tc_tiling_on_sc=False)


@functools.lru_cache(maxsize=None)
def _deg_kernel(J):
    @functools.partial(
        pl.kernel,
        out_type=jax.ShapeDtypeStruct((NC, NACC_DEG), jnp.float32),
        mesh=_mesh(),
        compiler_params=_SC_PARAMS,
        scratch_types=[
            pltpu.VMEM((J, K), jnp.int32),
            pltpu.VMEM((K,), jnp.float32),
            pltpu.VMEM((ROWS_DEG,), jnp.float32),
            pltpu.VMEM_SHARED((NACC_DEG,), jnp.float32),
        ],
    )
    def deg(dst_hbm, out_hbm, dst_v, ones_v, buf_v, acc):
        c = lax.axis_index("c")
        s = lax.axis_index("s")
        w = c * NS + s
        for i in range(K // 16):
            ones_v[pl.ds(i * 16, 16)] = jnp.full((16,), 1.0, jnp.float32)

        def zero_body(i, carry):
            buf_v[pl.ds(i * 16, 16)] = jnp.zeros((16,), jnp.float32)
            return carry

        lax.fori_loop(0, ROWS_DEG // 16, zero_body, 0)
        base = s * ROWS_DEG
        pltpu.sync_copy(buf_v, acc.at[pl.ds(base, ROWS_DEG)])
        pltpu.sync_copy(dst_hbm.at[w], dst_v)
        plsc.subcore_barrier()

        def body(j, carry):
            pltpu.sync_copy(ones_v, acc.at[dst_v.at[j]], add=True)
            return carry

        lax.fori_loop(0, J, body, 0)
        plsc.subcore_barrier()
        pltpu.sync_copy(acc.at[pl.ds(base, ROWS_DEG)], buf_v)
        pltpu.sync_copy(buf_v, out_hbm.at[c, pl.ds(base, ROWS_DEG)])

    return deg


def _gather_scatter_loop(y_ref, src_v, dst_v, bufs, acc, gsems, ssems, J):
    assert J % 4 == 0 and J >= 8
    G = J // 4

    def fire_g(j, b):
        pltpu.async_copy(y_ref.at[src_v.at[j]], bufs[b], gsems[b])

    def wait_g(j, b):
        pltpu.make_async_copy(y_ref.at[src_v.at[j]], bufs[b], gsems[b]).wait()

    def fire_s(j, b):
        pltpu.async_copy(bufs[b], acc.at[dst_v.at[j]], ssems[b], add=True)

    def wait_s(j, b):
        pltpu.make_async_copy(bufs[b], acc.at[dst_v.at[j]], ssems[b]).wait()

    def step(j, b, first, last):
        wait_g(j, b)
        fire_s(j, b)
        if not first:
            wait_s(j - 1, (b - 1) % 4)
        if not last:
            fire_g(j + 3, (b + 3) % 4)

    for b in range(3):
        fire_g(b, b)
    for b in range(4):
        step(b, b, first=(b == 0), last=False)

    def body(g, carry):
        j0 = 4 * g
        for b in range(4):
            step(j0 + b, b, first=False, last=False)
        return carry

    lax.fori_loop(1, G - 1, body, 0)
    for b in range(4):
        step(J - 4 + b, b, first=False, last=(b >= 1))
    wait_s(J - 1, 3)


def _zero_acc_slice(rows_a, acc, base, D):
    def zrow(i, carry):
        for jj in range(D // 16):
            rows_a[i, pl.ds(jj * 16, 16)] = jnp.zeros((16,), jnp.float32)
        return carry

    lax.fori_loop(0, K, zrow, 0)
    nfull = ROWS_PER_TILE // K
    rem = ROWS_PER_TILE - nfull * K
    for t in range(nfull):
        pltpu.sync_copy(rows_a, acc.at[pl.ds(base + t * K, K)])
    pltpu.sync_copy(rows_a.at[pl.ds(0, rem)],
                    acc.at[pl.ds(base + nfull * K, rem)])


def _copy_out_slice(rows_a, acc, out_ref, base, D):
    nfull = ROWS_PER_TILE // K
    rem = ROWS_PER_TILE - nfull * K
    for t in range(nfull):
        pltpu.sync_copy(acc.at[pl.ds(base + t * K, K)], rows_a)
        pltpu.sync_copy(rows_a, out_ref.at[pl.ds(base + t * K, K)])
    pltpu.sync_copy(acc.at[pl.ds(base + nfull * K, rem)],
                    rows_a.at[pl.ds(0, rem)])
    pltpu.sync_copy(rows_a.at[pl.ds(0, rem)],
                    out_ref.at[pl.ds(base + nfull * K, rem)])


@functools.lru_cache(maxsize=None)
def _agg1_kernel(J):
    @functools.partial(
        pl.kernel,
        out_type=jax.ShapeDtypeStruct((NC, NACC, HH), jnp.float32),
        mesh=_mesh(),
        compiler_params=_SC_PARAMS,
        scratch_types=[
            pltpu.VMEM((J, K), jnp.int32),
            pltpu.VMEM((J, K), jnp.int32),
            pltpu.VMEM((K, HH), jnp.float32),
            pltpu.VMEM((K, HH), jnp.float32),
            pltpu.VMEM((K, HH), jnp.float32),
            pltpu.VMEM((K, HH), jnp.float32),
            pltpu.VMEM_SHARED((NACC, HH), jnp.float32),
        ] + [pltpu.SemaphoreType.DMA] * 8,
    )
    def agg1(y_hbm, src_hbm, dst_hbm, out_hbm, src_v, dst_v, b0, b1, b2, b3,
             acc, *sems):
        c = lax.axis_index("c")
        s = lax.axis_index("s")
        base = s * ROWS_PER_TILE
        _zero_acc_slice(b0, acc, base, HH)
        pltpu.sync_copy(src_hbm.at[s], src_v)
        pltpu.sync_copy(dst_hbm.at[s], dst_v)
        plsc.subcore_barrier()
        _gather_scatter_loop(y_hbm.at[c], src_v, dst_v, [b0, b1, b2, b3],
                             acc, sems[:4], sems[4:], J)
        plsc.subcore_barrier()
        _copy_out_slice(b0, acc, out_hbm.at[c], base, HH)

    return agg1


@functools.lru_cache(maxsize=None)
def _agg2_kernel(J):
    @functools.partial(
        pl.kernel,
        out_type=jax.ShapeDtypeStruct((NC, NACC, C), jnp.float32),
        mesh=_mesh(),
        compiler_params=_SC_PARAMS,
        scratch_types=[
            pltpu.VMEM((J, K), jnp.int32),
            pltpu.VMEM((J, K), jnp.int32),
            pltpu.VMEM((K, C), jnp.float32),
            pltpu.VMEM((K, C), jnp.float32),
            pltpu.VMEM((K, C), jnp.float32),
            pltpu.VMEM((K, C), jnp.float32),
            pltpu.VMEM_SHARED((NACC, C), jnp.float32),
        ] + [pltpu.SemaphoreType.DMA] * 8,
    )
    def agg2(y_hbm, src_hbm, dst_hbm, out_hbm, src_v, dst_v, b0, b1, b2, b3,
             acc, *sems):
        c = lax.axis_index("c")
        s = lax.axis_index("s")
        w = c * NS + s
        base = s * ROWS_PER_TILE
        _zero_acc_slice(b0, acc, base, C)
        pltpu.sync_copy(src_hbm.at[w], src_v)
        pltpu.sync_copy(dst_hbm.at[w], dst_v)
        plsc.subcore_barrier()
        _gather_scatter_loop(y_hbm, src_v, dst_v, [b0, b1, b2, b3],
                             acc, sems[:4], sems[4:], J)
        plsc.subcore_barrier()
        _copy_out_slice(b0, acc, out_hbm.at[c], base, C)

    return agg2


def _prep(x, W1, degp):
    def body(x_ref, w_ref, deg_ref, yh_ref, dinv_ref):
        dinv = lax.rsqrt(deg_ref[0] + deg_ref[1] + 1.0)
        xw = jnp.dot(x_ref[...], w_ref[...], preferred_element_type=jnp.float32)
        y = xw * dinv
        yh_ref[0] = y[:, :HH]
        yh_ref[1] = y[:, HH:]
        dinv_ref[...] = dinv

    return pl.pallas_call(
        body,
        out_shape=(
            jax.ShapeDtypeStruct((NC, N, HH), jnp.float32),
            jax.ShapeDtypeStruct((N, 1), jnp.float32),
        ),
    )(x, W1, degp)


def _mid(agg1, yh, dinv, b1, W2):
    def body(agg_ref, yh_ref, dinv_ref, b_ref, w_ref, o_ref):
        dinv = dinv_ref[...]
        acat = jnp.concatenate([agg_ref[0, :N], agg_ref[1, :N]], axis=1)
        ycat = jnp.concatenate([yh_ref[0], yh_ref[1]], axis=1)
        pre = dinv * (acat + ycat) + b_ref[...][None, :]
        h = jnp.maximum(pre, 0.0)
        o_ref[...] = dinv * jnp.dot(h, w_ref[...],
                                    preferred_element_type=jnp.float32)

    return pl.pallas_call(
        body,
        out_shape=jax.ShapeDtypeStruct((N, C), jnp.float32),
    )(agg1, yh, dinv, b1, W2)


def _final(agg2, y2, dinv, b2):
    def body(agg_ref, y_ref, dinv_ref, b_ref, o_ref):
        o_ref[...] = (dinv_ref[...] * (agg_ref[0, :N] + agg_ref[1, :N]
                                       + y_ref[...])
                      + b_ref[...][None, :])

    return pl.pallas_call(
        body,
        out_shape=jax.ShapeDtypeStruct((N, C), jnp.float32),
    )(agg2, y2, dinv, b2)


def _pad_edges(ei, shards, J):
    E = ei.shape[1]
    pad = shards * J * K - E
    ar = jnp.arange(pad, dtype=jnp.int32)
    pad_src = (ar * 7993) % N
    pad_dst = N + (ar % (NACC - N))
    src = jnp.concatenate([ei[0], pad_src]).reshape(shards, J, K)
    dst = jnp.concatenate([ei[1], pad_dst]).reshape(shards, J, K)
    return src, dst


def _num_chunks(E, shards):
    J = -(-E // (shards * K))
    return -(-J // 4) * 4


def kernel(x, edge_index, W1, b1, W2, b2):
    E = edge_index.shape[1]
    ei = edge_index.astype(jnp.int32)

    J1 = _num_chunks(E, NS)
    J2 = _num_chunks(E, NW)
    src1, dst1 = _pad_edges(ei, NS, J1)
    src2, dst2 = _pad_edges(ei, NW, J2)

    deg = _deg_kernel(J2)(dst2)
    degp = deg[:, :N, None]
    yh, dinv = _prep(x, W1, degp)
    agg1 = _agg1_kernel(J1)(yh, src1, dst1)
    y2 = _mid(agg1, yh, dinv, b1, W2)
    agg2 = _agg2_kernel(J2)(y2, src2, dst2)
    return _final(agg2, y2, dinv, b2)


# speedup vs baseline: 41.9316x; 1.0413x over previous
import functools

import jax
import jax.numpy as jnp
from jax import lax
from jax.experimental import pallas as pl
from jax.experimental.pallas import tpu as pltpu
from jax.experimental.pallas import tpu_sc as plsc

N = 10000
H = 128
HH = H // 2
C = 16

NC = 2
NS = 16
NW = NC * NS
K = 128
M1 = 2
M2 = 10

ROWS_PER_TILE = 632
NACC = NS * ROWS_PER_TILE
ROWS_DEG = 640
NACC_DEG = NS * ROWS_DEG


def _mesh():
    return plsc.VectorSubcoreMesh(core_axis_name="c", subcore_axis_name="s")


_SC_PARAMS = pltpu.CompilerParams(use_tc_tiling_on_sc=False)


@functools.lru_cache(maxsize=None)
def _deg_kernel(J):
    @functools.partial(
        pl.kernel,
        out_type=jax.ShapeDtypeStruct((NC, NACC_DEG), jnp.float32),
        mesh=_mesh(),
        compiler_params=_SC_PARAMS,
        scratch_types=[
            pltpu.VMEM((J, K), jnp.int32),
            pltpu.VMEM((K,), jnp.float32),
            pltpu.VMEM((ROWS_DEG,), jnp.float32),
            pltpu.VMEM_SHARED((NACC_DEG,), jnp.float32),
        ],
    )
    def deg(dst_hbm, out_hbm, dst_v, ones_v, buf_v, acc):
        c = lax.axis_index("c")
        s = lax.axis_index("s")
        w = c * NS + s
        for i in range(K // 16):
            ones_v[pl.ds(i * 16, 16)] = jnp.full((16,), 1.0, jnp.float32)

        def zero_body(i, carry):
            buf_v[pl.ds(i * 16, 16)] = jnp.zeros((16,), jnp.float32)
            return carry

        lax.fori_loop(0, ROWS_DEG // 16, zero_body, 0)
        base = s * ROWS_DEG
        pltpu.sync_copy(buf_v, acc.at[pl.ds(base, ROWS_DEG)])
        pltpu.sync_copy(dst_hbm.at[w], dst_v)
        plsc.subcore_barrier()

        def body(j, carry):
            pltpu.sync_copy(ones_v, acc.at[dst_v.at[j]], add=True)
            return carry

        lax.fori_loop(0, J, body, 0)
        plsc.subcore_barrier()
        pltpu.sync_copy(acc.at[pl.ds(base, ROWS_DEG)], buf_v)
        pltpu.sync_copy(buf_v, out_hbm.at[c, pl.ds(base, ROWS_DEG)])

    return deg


def _gather_scatter_loop(y_ref, src_v, dst_v, bufs, acc, gsems, ssems, J, M):
    assert J % (4 * M) == 0 and J // M >= 8
    G = J // M // 4

    def fire_g(g, b):
        pltpu.async_copy(y_ref.at[src_v.at[g]], bufs[b], gsems[b])

    def wait_g(g, b):
        pltpu.make_async_copy(y_ref.at[src_v.at[g]], bufs[b],
                              gsems[b]).wait()

    def fire_s(g, b):
        pltpu.async_copy(bufs[b], acc.at[dst_v.at[g]], ssems[b], add=True)

    def wait_s(g, b):
        pltpu.make_async_copy(bufs[b], acc.at[dst_v.at[g]], ssems[b]).wait()

    SJ = J // M

    def step(g, b, first, last):
        wait_g(g, b)
        fire_s(g, b)
        if not first:
            wait_s(g - 1, (b - 1) % 4)
        if not last:
            fire_g(g + 3, (b + 3) % 4)

    for b in range(3):
        fire_g(b, b)
    for b in range(4):
        step(b, b, first=(b == 0), last=False)

    def body(t, carry):
        g0 = 4 * t
        for b in range(4):
            step(g0 + b, b, first=False, last=False)
        return carry

    lax.fori_loop(1, G - 1, body, 0)
    for b in range(4):
        step(SJ - 4 + b, b, first=False, last=(b >= 1))
    wait_s(SJ - 1, 3)


def _zero_acc_slice(rows_a, acc, base, D):
    def zrow(i, carry):
        for jj in range(D // 16):
            rows_a[i, pl.ds(jj * 16, 16)] = jnp.zeros((16,), jnp.float32)
        return carry

    lax.fori_loop(0, K, zrow, 0)
    nfull = ROWS_PER_TILE // K
    rem = ROWS_PER_TILE - nfull * K
    for t in range(nfull):
        pltpu.sync_copy(rows_a, acc.at[pl.ds(base + t * K, K)])
    pltpu.sync_copy(rows_a.at[pl.ds(0, rem)],
                    acc.at[pl.ds(base + nfull * K, rem)])


def _copy_out_slice(rows_a, acc, out_ref, base, D):
    nfull = ROWS_PER_TILE // K
    rem = ROWS_PER_TILE - nfull * K
    for t in range(nfull):
        pltpu.sync_copy(acc.at[pl.ds(base + t * K, K)], rows_a)
        pltpu.sync_copy(rows_a, out_ref.at[pl.ds(base + t * K, K)])
    pltpu.sync_copy(acc.at[pl.ds(base + nfull * K, rem)],
                    rows_a.at[pl.ds(0, rem)])
    pltpu.sync_copy(rows_a.at[pl.ds(0, rem)],
                    out_ref.at[pl.ds(base + nfull * K, rem)])


@functools.lru_cache(maxsize=None)
def _agg1_kernel(J):
    @functools.partial(
        pl.kernel,
        out_type=jax.ShapeDtypeStruct((NC, NACC, HH), jnp.float32),
        mesh=_mesh(),
        compiler_params=_SC_PARAMS,
        scratch_types=[
            pltpu.VMEM((J // M1 // 2, M1 * K), jnp.int32),
            pltpu.VMEM((J // M1 // 2, M1 * K), jnp.int32),
            pltpu.VMEM((M1 * K, HH), jnp.float32),
            pltpu.VMEM((M1 * K, HH), jnp.float32),
            pltpu.VMEM((M1 * K, HH), jnp.float32),
            pltpu.VMEM((M1 * K, HH), jnp.float32),
            pltpu.VMEM_SHARED((NACC, HH), jnp.float32),
        ] + [pltpu.SemaphoreType.DMA] * 8,
    )
    def agg1(y_hbm, src_hbm, dst_hbm, out_hbm, src_v, dst_v, b0, b1, b2, b3,
             acc, *sems):
        c = lax.axis_index("c")
        s = lax.axis_index("s")
        base = s * ROWS_PER_TILE
        _zero_acc_slice(b0.at[pl.ds(0, K)], acc, base, HH)
        plsc.subcore_barrier()
        SJH = J // M1 // 2
        for h in range(2):
            pltpu.sync_copy(src_hbm.at[s, pl.ds(h * SJH, SJH)], src_v)
            pltpu.sync_copy(dst_hbm.at[s, pl.ds(h * SJH, SJH)], dst_v)
            _gather_scatter_loop(y_hbm.at[c], src_v, dst_v, [b0, b1, b2, b3],
                                 acc, sems[:4], sems[4:], J // 2, M1)
        plsc.subcore_barrier()
        _copy_out_slice(b0.at[pl.ds(0, K)], acc, out_hbm.at[c], base, HH)

    return agg1


@functools.lru_cache(maxsize=None)
def _agg2_kernel(J):
    @functools.partial(
        pl.kernel,
        out_type=jax.ShapeDtypeStruct((NC, NACC, C), jnp.float32),
        mesh=_mesh(),
        compiler_params=_SC_PARAMS,
        scratch_types=[
            pltpu.VMEM((J // M2, M2 * K), jnp.int32),
            pltpu.VMEM((J // M2, M2 * K), jnp.int32),
            pltpu.VMEM((M2 * K, C), jnp.float32),
            pltpu.VMEM((M2 * K, C), jnp.float32),
            pltpu.VMEM((M2 * K, C), jnp.float32),
            pltpu.VMEM((M2 * K, C), jnp.float32),
            pltpu.VMEM_SHARED((NACC, C), jnp.float32),
        ] + [pltpu.SemaphoreType.DMA] * 8,
    )
    def agg2(y_hbm, src_hbm, dst_hbm, out_hbm, src_v, dst_v, b0, b1, b2, b3,
             acc, *sems):
        c = lax.axis_index("c")
        s = lax.axis_index("s")
        w = c * NS + s
        base = s * ROWS_PER_TILE
        _zero_acc_slice(b0.at[pl.ds(0, K)], acc, base, C)
        pltpu.sync_copy(src_hbm.at[w], src_v)
        pltpu.sync_copy(dst_hbm.at[w], dst_v)
        plsc.subcore_barrier()
        _gather_scatter_loop(y_hbm, src_v, dst_v, [b0, b1, b2, b3],
                             acc, sems[:4], sems[4:], J, M2)
        plsc.subcore_barrier()
        _copy_out_slice(b0.at[pl.ds(0, K)], acc, out_hbm.at[c], base, C)

    return agg2


def _prep(x, W1, degp):
    def body(x_ref, w_ref, deg_ref, yh_ref, dinv_ref):
        dinv = lax.rsqrt(deg_ref[0] + deg_ref[1] + 1.0)
        xw = jnp.dot(x_ref[...], w_ref[...], preferred_element_type=jnp.float32)
        y = xw * dinv
        yh_ref[0] = y[:, :HH]
        yh_ref[1] = y[:, HH:]
        dinv_ref[...] = dinv

    return pl.pallas_call(
        body,
        out_shape=(
            jax.ShapeDtypeStruct((NC, N, HH), jnp.float32),
            jax.ShapeDtypeStruct((N, 1), jnp.float32),
        ),
    )(x, W1, degp)


def _mid(agg1, yh, dinv, b1, W2):
    def body(agg_ref, yh_ref, dinv_ref, b_ref, w_ref, o_ref):
        dinv = dinv_ref[...]
        acat = jnp.concatenate([agg_ref[0, :N], agg_ref[1, :N]], axis=1)
        ycat = jnp.concatenate([yh_ref[0], yh_ref[1]], axis=1)
        pre = dinv * (acat + ycat) + b_ref[...][None, :]
        h = jnp.maximum(pre, 0.0)
        o_ref[...] = dinv * jnp.dot(h, w_ref[...],
                                    preferred_element_type=jnp.float32)

    return pl.pallas_call(
        body,
        out_shape=jax.ShapeDtypeStruct((N, C), jnp.float32),
    )(agg1, yh, dinv, b1, W2)


def _final(agg2, y2, dinv, b2):
    def body(agg_ref, y_ref, dinv_ref, b_ref, o_ref):
        o_ref[...] = (dinv_ref[...] * (agg_ref[0, :N] + agg_ref[1, :N]
                                       + y_ref[...])
                      + b_ref[...][None, :])

    return pl.pallas_call(
        body,
        out_shape=jax.ShapeDtypeStruct((N, C), jnp.float32),
    )(agg2, y2, dinv, b2)


def _pad_edges(ei, shards, J, M=None):
    E = ei.shape[1]
    pad = shards * J * K - E
    ar = jnp.arange(pad, dtype=jnp.int32)
    pad_src = (ar * 7993) % N
    pad_dst = N + (ar % (NACC - N))
    if M is None:
        shape = (shards, J, K)
    else:
        shape = (shards, J // M, M * K)
    src = jnp.concatenate([ei[0], pad_src]).reshape(shape)
    dst = jnp.concatenate([ei[1], pad_dst]).reshape(shape)
    return src, dst


def _num_chunks(E, shards, mult):
    J = -(-E // (shards * K))
    return -(-J // mult) * mult


def kernel(x, edge_index, W1, b1, W2, b2):
    E = edge_index.shape[1]
    ei = edge_index.astype(jnp.int32)

    J1 = _num_chunks(E, NS, 4 * M1)
    J2 = _num_chunks(E, NW, 4 * M2)
    src1, dst1 = _pad_edges(ei, NS, J1, M1)
    src2, dst2 = _pad_edges(ei, NW, J2, M2)

    _, dstd = _pad_edges(ei, NW, J2)
    deg = _deg_kernel(J2)(dstd)
    degp = deg[:, :N, None]
    yh, dinv = _prep(x, W1, degp)
    agg1 = _agg1_kernel(J1)(yh, src1, dst1)
    y2 = _mid(agg1, yh, dinv, b1, W2)
    agg2 = _agg2_kernel(J2)(y2, src2, dst2)
    return _final(agg2, y2, dinv, b2)


# speedup vs baseline: 41.9329x; 1.0000x over previous
import functools

import jax
import jax.numpy as jnp
from jax import lax
from jax.experimental import pallas as pl
from jax.experimental.pallas import tpu as pltpu
from jax.experimental.pallas import tpu_sc as plsc

N = 10000
H = 128
HH = H // 2
C = 16

NC = 2
NS = 16
NW = NC * NS
K = 128
M1 = 2
M2 = 10

ROWS_PER_TILE = 632
NACC = NS * ROWS_PER_TILE
ROWS_DEG = 640
NACC_DEG = NS * ROWS_DEG


def _mesh():
    return plsc.VectorSubcoreMesh(core_axis_name="c", subcore_axis_name="s")


_SC_PARAMS = pltpu.CompilerParams(use_tc_tiling_on_sc=False)


@functools.lru_cache(maxsize=None)
def _deg_kernel(J):
    @functools.partial(
        pl.kernel,
        out_type=jax.ShapeDtypeStruct((NC, NACC_DEG), jnp.float32),
        mesh=_mesh(),
        scratch_types=[
            pltpu.VMEM((J, K), jnp.int32),
            pltpu.VMEM((K,), jnp.float32),
            pltpu.VMEM((ROWS_DEG,), jnp.float32),
            pltpu.VMEM_SHARED((NACC_DEG,), jnp.float32),
        ],
    )
    def deg(dst_hbm, out_hbm, dst_v, ones_v, buf_v, acc):
        c = lax.axis_index("c")
        s = lax.axis_index("s")
        w = c * NS + s
        for i in range(K // 16):
            ones_v[pl.ds(i * 16, 16)] = jnp.full((16,), 1.0, jnp.float32)

        def zero_body(i, carry):
            buf_v[pl.ds(i * 16, 16)] = jnp.zeros((16,), jnp.float32)
            return carry

        lax.fori_loop(0, ROWS_DEG // 16, zero_body, 0)
        base = s * ROWS_DEG
        pltpu.sync_copy(buf_v, acc.at[pl.ds(base, ROWS_DEG)])
        pltpu.sync_copy(dst_hbm.at[w], dst_v)
        plsc.subcore_barrier()

        def body(j, carry):
            pltpu.sync_copy(ones_v, acc.at[dst_v.at[j]], add=True)
            return carry

        lax.fori_loop(0, J, body, 0)
        plsc.subcore_barrier()
        pltpu.sync_copy(acc.at[pl.ds(base, ROWS_DEG)], buf_v)
        pltpu.sync_copy(buf_v, out_hbm.at[c, pl.ds(base, ROWS_DEG)])

    return deg


def _gather_scatter_loop(y_ref, src_v, dst_v, bufs, acc, gsems, ssems, J, M):
    assert J % (4 * M) == 0 and J // M >= 8
    G = J // M // 4

    def fire_g(g, b):
        pltpu.async_copy(y_ref.at[src_v.at[g]], bufs[b], gsems[b])

    def wait_g(g, b):
        pltpu.make_async_copy(y_ref.at[src_v.at[g]], bufs[b],
                              gsems[b]).wait()

    def fire_s(g, b):
        pltpu.async_copy(bufs[b], acc.at[dst_v.at[g]], ssems[b], add=True)

    def wait_s(g, b):
        pltpu.make_async_copy(bufs[b], acc.at[dst_v.at[g]], ssems[b]).wait()

    SJ = J // M

    def step(g, b, first, last):
        wait_g(g, b)
        fire_s(g, b)
        if not first:
            wait_s(g - 1, (b - 1) % 4)
        if not last:
            fire_g(g + 3, (b + 3) % 4)

    for b in range(3):
        fire_g(b, b)
    for b in range(4):
        step(b, b, first=(b == 0), last=False)

    def body(t, carry):
        g0 = 4 * t
        for b in range(4):
            step(g0 + b, b, first=False, last=False)
        return carry

    lax.fori_loop(1, G - 1, body, 0)
    for b in range(4):
        step(SJ - 4 + b, b, first=False, last=(b >= 1))
    wait_s(SJ - 1, 3)


def _zero_acc_slice(rows_a, acc, base, D):
    def zrow(i, carry):
        for jj in range(D // 16):
            rows_a[i, pl.ds(jj * 16, 16)] = jnp.zeros((16,), jnp.float32)
        return carry

    lax.fori_loop(0, K, zrow, 0)
    nfull = ROWS_PER_TILE // K
    rem = ROWS_PER_TILE - nfull * K
    for t in range(nfull):
        pltpu.sync_copy(rows_a, acc.at[pl.ds(base + t * K, K)])
    pltpu.sync_copy(rows_a.at[pl.ds(0, rem)],
                    acc.at[pl.ds(base + nfull * K, rem)])


def _copy_out_slice(rows_a, acc, out_ref, base, D):
    nfull = ROWS_PER_TILE // K
    rem = ROWS_PER_TILE - nfull * K
    for t in range(nfull):
        pltpu.sync_copy(acc.at[pl.ds(base + t * K, K)], rows_a)
        pltpu.sync_copy(rows_a, out_ref.at[pl.ds(base + t * K, K)])
    pltpu.sync_copy(acc.at[pl.ds(base + nfull * K, rem)],
                    rows_a.at[pl.ds(0, rem)])
    pltpu.sync_copy(rows_a.at[pl.ds(0, rem)],
                    out_ref.at[pl.ds(base + nfull * K, rem)])


@functools.lru_cache(maxsize=None)
def _agg1_kernel(J):
    @functools.partial(
        pl.kernel,
        out_type=jax.ShapeDtypeStruct((NC, NACC, HH), jnp.float32),
        mesh=_mesh(),
        compiler_params=_SC_PARAMS,
        scratch_types=[
            pltpu.VMEM((J // M1 // 2, M1 * K), jnp.int32),
            pltpu.VMEM((J // M1 // 2, M1 * K), jnp.int32),
            pltpu.VMEM((M1 * K, HH), jnp.float32),
            pltpu.VMEM((M1 * K, HH), jnp.float32),
            pltpu.VMEM((M1 * K, HH), jnp.float32),
            pltpu.VMEM((M1 * K, HH), jnp.float32),
            pltpu.VMEM_SHARED((NACC, HH), jnp.float32),
        ] + [pltpu.SemaphoreType.DMA] * 8,
    )
    def agg1(y_hbm, src_hbm, dst_hbm, out_hbm, src_v, dst_v, b0, b1, b2, b3,
             acc, *sems):
        c = lax.axis_index("c")
        s = lax.axis_index("s")
        base = s * ROWS_PER_TILE
        _zero_acc_slice(b0.at[pl.ds(0, K)], acc, base, HH)
        plsc.subcore_barrier()
        SJH = J // M1 // 2
        for h in range(2):
            pltpu.sync_copy(src_hbm.at[s, pl.ds(h * SJH, SJH)], src_v)
            pltpu.sync_copy(dst_hbm.at[s, pl.ds(h * SJH, SJH)], dst_v)
            _gather_scatter_loop(y_hbm.at[c], src_v, dst_v, [b0, b1, b2, b3],
                                 acc, sems[:4], sems[4:], J // 2, M1)
        plsc.subcore_barrier()
        _copy_out_slice(b0.at[pl.ds(0, K)], acc, out_hbm.at[c], base, HH)

    return agg1


@functools.lru_cache(maxsize=None)
def _agg2_kernel(J):
    @functools.partial(
        pl.kernel,
        out_type=jax.ShapeDtypeStruct((NC, NACC, C), jnp.float32),
        mesh=_mesh(),
        compiler_params=_SC_PARAMS,
        scratch_types=[
            pltpu.VMEM((J // M2, M2 * K), jnp.int32),
            pltpu.VMEM((J // M2, M2 * K), jnp.int32),
            pltpu.VMEM((M2 * K, C), jnp.float32),
            pltpu.VMEM((M2 * K, C), jnp.float32),
            pltpu.VMEM((M2 * K, C), jnp.float32),
            pltpu.VMEM((M2 * K, C), jnp.float32),
            pltpu.VMEM_SHARED((NACC, C), jnp.float32),
        ] + [pltpu.SemaphoreType.DMA] * 8,
    )
    def agg2(y_hbm, src_hbm, dst_hbm, out_hbm, src_v, dst_v, b0, b1, b2, b3,
             acc, *sems):
        c = lax.axis_index("c")
        s = lax.axis_index("s")
        w = c * NS + s
        base = s * ROWS_PER_TILE
        _zero_acc_slice(b0.at[pl.ds(0, K)], acc, base, C)
        pltpu.sync_copy(src_hbm.at[w], src_v)
        pltpu.sync_copy(dst_hbm.at[w], dst_v)
        plsc.subcore_barrier()
        _gather_scatter_loop(y_hbm, src_v, dst_v, [b0, b1, b2, b3],
                             acc, sems[:4], sems[4:], J, M2)
        plsc.subcore_barrier()
        _copy_out_slice(b0.at[pl.ds(0, K)], acc, out_hbm.at[c], base, C)

    return agg2


def _matmul1(x, W1):
    def body(x_ref, w_ref, o_ref):
        o_ref[...] = jnp.dot(x_ref[...], w_ref[...],
                             preferred_element_type=jnp.float32)

    return pl.pallas_call(
        body,
        out_shape=jax.ShapeDtypeStruct((N, H), jnp.float32),
    )(x, W1)


def _scale(xw, degp):
    def body(xw_ref, deg_ref, yh_ref, dinv_ref):
        dinv = lax.rsqrt(deg_ref[0] + deg_ref[1] + 1.0)
        y = xw_ref[...] * dinv
        yh_ref[0] = y[:, :HH]
        yh_ref[1] = y[:, HH:]
        dinv_ref[...] = dinv

    return pl.pallas_call(
        body,
        out_shape=(
            jax.ShapeDtypeStruct((NC, N, HH), jnp.float32),
            jax.ShapeDtypeStruct((N, 1), jnp.float32),
        ),
    )(xw, degp)


def _mid(agg1, yh, dinv, b1, W2):
    def body(agg_ref, yh_ref, dinv_ref, b_ref, w_ref, o_ref):
        dinv = dinv_ref[...]
        acat = jnp.concatenate([agg_ref[0, :N], agg_ref[1, :N]], axis=1)
        ycat = jnp.concatenate([yh_ref[0], yh_ref[1]], axis=1)
        pre = dinv * (acat + ycat) + b_ref[...][None, :]
        h = jnp.maximum(pre, 0.0)
        o_ref[...] = dinv * jnp.dot(h, w_ref[...],
                                    preferred_element_type=jnp.float32)

    return pl.pallas_call(
        body,
        out_shape=jax.ShapeDtypeStruct((N, C), jnp.float32),
    )(agg1, yh, dinv, b1, W2)


def _final(agg2, y2, dinv, b2):
    def body(agg_ref, y_ref, dinv_ref, b_ref, o_ref):
        o_ref[...] = (dinv_ref[...] * (agg_ref[0, :N] + agg_ref[1, :N]
                                       + y_ref[...])
                      + b_ref[...][None, :])

    return pl.pallas_call(
        body,
        out_shape=jax.ShapeDtypeStruct((N, C), jnp.float32),
    )(agg2, y2, dinv, b2)


def _pad_edges(ei, shards, J, M=None):
    E = ei.shape[1]
    pad = shards * J * K - E
    ar = jnp.arange(pad, dtype=jnp.int32)
    pad_src = (ar * 7993) % N
    pad_dst = N + (ar % (NACC - N))
    if M is None:
        shape = (shards, J, K)
    else:
        shape = (shards, J // M, M * K)
    src = jnp.concatenate([ei[0], pad_src]).reshape(shape)
    dst = jnp.concatenate([ei[1], pad_dst]).reshape(shape)
    return src, dst


def _num_chunks(E, shards, mult):
    J = -(-E // (shards * K))
    return -(-J // mult) * mult


def kernel(x, edge_index, W1, b1, W2, b2):
    E = edge_index.shape[1]
    ei = edge_index.astype(jnp.int32)

    J1 = _num_chunks(E, NS, 4 * M1)
    J2 = _num_chunks(E, NW, 4 * M2)
    src1, dst1 = _pad_edges(ei, NS, J1, M1)
    src2, dst2 = _pad_edges(ei, NW, J2, M2)

    _, dstd = _pad_edges(ei, NW, J2)
    xw = _matmul1(x, W1)
    deg = _deg_kernel(J2)(dstd)
    degp = deg[:, :N, None]
    yh, dinv = _scale(xw, degp)
    agg1 = _agg1_kernel(J1)(yh, src1, dst1)
    y2 = _mid(agg1, yh, dinv, b1, W2)
    agg2 = _agg2_kernel(J2)(y2, src2, dst2)
    return _final(agg2, y2, dinv, b2)
